# Initial kernel scaffold; baseline (speedup 1.0000x reference)
import jax
import jax.numpy as jnp
from jax.experimental import pallas as pl


def kernel(positions, W, b, gamma, beta):
    raise NotImplementedError("write your pallas kernel here")


# speedup vs baseline: 1.9996x; 1.9996x over previous
import numpy as np
import jax
import jax.numpy as jnp
from jax.experimental import pallas as pl

D_MODEL = 1024
D_HALF = D_MODEL // 2
_SPINE = (0.0, 2.0, 4.0, 12.0, 36.0, 104.0, 304.0, 888.0, 2592.0, 7568.0)
_TOK_BLOCK = 512


def _pe_kernel(pos_ref, df_ref, ph_ref, wt_ref, b_ref, g_ref, be_ref, out_ref):
    pos = pos_ref[...]
    df = df_ref[...]
    ph = ph_ref[...]
    arg = pos * df + ph
    pe = jnp.sin(arg)

    lvl = jnp.zeros_like(pos)
    floor = jnp.zeros_like(pos)
    for s in _SPINE:
        ge = pos >= s
        lvl = lvl + ge.astype(jnp.float32)
        floor = jnp.where(ge, s, floor)
    ceil = pos
    for s in _SPINE[:0:-1]:
        ceil = jnp.where(pos < s, s, ceil)
    left = pos - floor
    right = ceil - pos

    h = (left * wt_ref[0:1, :] + right * wt_ref[1:2, :] + lvl * wt_ref[2:3, :]
         + b_ref[...])
    mu = jnp.mean(h, axis=1, keepdims=True)
    d = h - mu
    var = jnp.mean(d * d, axis=1, keepdims=True)
    hn = d * jax.lax.rsqrt(var + 1e-5) * g_ref[...] + be_ref[...]
    lat = 0.5 * hn * (1.0 + jax.lax.erf(hn * np.float32(1.0 / np.sqrt(2.0))))

    out_ref[:, :D_HALF] = pe
    out_ref[:, D_HALF:] = lat


def kernel(positions, W, b, gamma, beta):
    B, S = positions.shape
    N = B * S
    d_half = W.shape[0]
    pos_f = positions.reshape(N, 1).astype(jnp.float32)

    div = np.exp(np.arange(0, d_half, 2, dtype=np.float64)
                 * -(np.log(10000.0) / d_half))
    df = jnp.asarray(np.repeat(div, 2).reshape(1, d_half), dtype=jnp.float32)
    ph = jnp.asarray(np.tile(np.array([0.0, np.pi / 2.0]), d_half // 2)
                     .reshape(1, d_half), dtype=jnp.float32)
    wt = W.T
    b2 = b.reshape(1, d_half)
    g2 = gamma.reshape(1, d_half)
    be2 = beta.reshape(1, d_half)

    T = _TOK_BLOCK
    grid = (N // T,)
    out = pl.pallas_call(
        _pe_kernel,
        grid=grid,
        in_specs=[
            pl.BlockSpec((T, 1), lambda i: (i, 0)),
            pl.BlockSpec((1, d_half), lambda i: (0, 0)),
            pl.BlockSpec((1, d_half), lambda i: (0, 0)),
            pl.BlockSpec((3, d_half), lambda i: (0, 0)),
            pl.BlockSpec((1, d_half), lambda i: (0, 0)),
            pl.BlockSpec((1, d_half), lambda i: (0, 0)),
            pl.BlockSpec((1, d_half), lambda i: (0, 0)),
        ],
        out_specs=pl.BlockSpec((T, 2 * d_half), lambda i: (i, 0)),
        out_shape=jax.ShapeDtypeStruct((N, 2 * d_half), jnp.float32),
    )(pos_f, df, ph, wt, b2, g2, be2)
    return out.reshape(B, S, 2 * d_half)


# speedup vs baseline: 4.4539x; 2.2274x over previous
import numpy as np
import jax
import jax.numpy as jnp
from jax.experimental import pallas as pl

D_MODEL = 1024
D_HALF = D_MODEL // 2
_SPINE = (0.0, 2.0, 4.0, 12.0, 36.0, 104.0, 304.0, 888.0, 2592.0, 7568.0)
_TOK_BLOCK = 512


_S1 = np.float32(3.14158198)
_S3 = np.float32(-5.1671413)
_S5 = np.float32(2.54188707)
_S7 = np.float32(-0.55460885)


def _pe_kernel(pos_ref, df_ref, ph_ref, wt_ref, b_ref, g_ref, be_ref, out_ref):
    pos = pos_ref[...]
    df = df_ref[...]
    ph = ph_ref[...]
    v = pos * df + ph
    n = jnp.floor(v + 0.5)
    t = v - n
    t2 = t * t
    p = t * (_S1 + t2 * (_S3 + t2 * (_S5 + t2 * _S7)))
    m = n * 0.5
    s = m - jnp.floor(m)
    pe = p - 4.0 * (s * p)

    lvl = jnp.zeros_like(pos)
    floor = jnp.zeros_like(pos)
    for s in _SPINE:
        ge = pos >= s
        lvl = lvl + ge.astype(jnp.float32)
        floor = jnp.where(ge, s, floor)
    ceil = pos
    for s in _SPINE[:0:-1]:
        ceil = jnp.where(pos < s, s, ceil)
    left = pos - floor
    right = ceil - pos

    h = (left * wt_ref[0:1, :] + right * wt_ref[1:2, :] + lvl * wt_ref[2:3, :]
         + b_ref[...])
    mu = jnp.mean(h, axis=1, keepdims=True)
    d = h - mu
    var = jnp.mean(d * d, axis=1, keepdims=True)
    hn = d * jax.lax.rsqrt(var + 1e-5) * g_ref[...] + be_ref[...]
    lat = 0.5 * hn * (1.0 + jax.lax.erf(hn * np.float32(1.0 / np.sqrt(2.0))))

    out_ref[:, :D_HALF] = pe
    out_ref[:, D_HALF:] = lat


def kernel(positions, W, b, gamma, beta):
    B, S = positions.shape
    N = B * S
    d_half = W.shape[0]
    pos_f = positions.reshape(N, 1).astype(jnp.float32)

    div = np.exp(np.arange(0, d_half, 2, dtype=np.float64)
                 * -(np.log(10000.0) / d_half))
    df = jnp.asarray(np.repeat(div / np.pi, 2).reshape(1, d_half),
                     dtype=jnp.float32)
    ph = jnp.asarray(np.tile(np.array([0.0, 0.5]), d_half // 2)
                     .reshape(1, d_half), dtype=jnp.float32)
    wt = W.T
    b2 = b.reshape(1, d_half)
    g2 = gamma.reshape(1, d_half)
    be2 = beta.reshape(1, d_half)

    T = _TOK_BLOCK
    grid = (N // T,)
    out = pl.pallas_call(
        _pe_kernel,
        grid=grid,
        in_specs=[
            pl.BlockSpec((T, 1), lambda i: (i, 0)),
            pl.BlockSpec((1, d_half), lambda i: (0, 0)),
            pl.BlockSpec((1, d_half), lambda i: (0, 0)),
            pl.BlockSpec((3, d_half), lambda i: (0, 0)),
            pl.BlockSpec((1, d_half), lambda i: (0, 0)),
            pl.BlockSpec((1, d_half), lambda i: (0, 0)),
            pl.BlockSpec((1, d_half), lambda i: (0, 0)),
        ],
        out_specs=pl.BlockSpec((T, 2 * d_half), lambda i: (i, 0)),
        out_shape=jax.ShapeDtypeStruct((N, 2 * d_half), jnp.float32),
    )(pos_f, df, ph, wt, b2, g2, be2)
    return out.reshape(B, S, 2 * d_half)


# speedup vs baseline: 4.7038x; 1.0561x over previous
import numpy as np
import jax
import jax.numpy as jnp
from jax.experimental import pallas as pl

D_MODEL = 1024
D_HALF = D_MODEL // 2
_SPINE = (0.0, 2.0, 4.0, 12.0, 36.0, 104.0, 304.0, 888.0, 2592.0, 7568.0)
_TOK_BLOCK = 512


_S1 = np.float32(3.141527043972124)
_S3 = np.float32(-5.1663903685742305)
_S5 = np.float32(2.542671830189423)
_S7 = np.float32(-0.5818045120989042)
_S9 = np.float32(0.06400176254731299)


def _pe_kernel(pos_ref, df_ref, ph_ref, wt_ref, b_ref, g_ref, be_ref, out_ref):
    pos = pos_ref[...]
    df = df_ref[...]
    ph = ph_ref[...]
    u = pos * df + ph
    f = u - jnp.floor(u)
    t = 2.0 * f - 1.0
    t2 = t * t
    pe = t * (_S1 + t2 * (_S3 + t2 * (_S5 + t2 * (_S7 + t2 * _S9))))

    lvl = jnp.zeros_like(pos)
    floor = jnp.zeros_like(pos)
    for s in _SPINE:
        ge = pos >= s
        lvl = lvl + ge.astype(jnp.float32)
        floor = jnp.where(ge, s, floor)
    ceil = pos
    for s in _SPINE[:0:-1]:
        ceil = jnp.where(pos < s, s, ceil)
    left = pos - floor
    right = ceil - pos

    h = (left * wt_ref[0:1, :] + right * wt_ref[1:2, :] + lvl * wt_ref[2:3, :]
         + b_ref[...])
    mu = jnp.mean(h, axis=1, keepdims=True)
    d = h - mu
    var = jnp.mean(d * d, axis=1, keepdims=True)
    hn = d * jax.lax.rsqrt(var + 1e-5) * g_ref[...] + be_ref[...]
    lat = 0.5 * hn * (1.0 + jax.lax.erf(hn * np.float32(1.0 / np.sqrt(2.0))))

    out_ref[:, :D_HALF] = pe
    out_ref[:, D_HALF:] = lat


def kernel(positions, W, b, gamma, beta):
    B, S = positions.shape
    N = B * S
    d_half = W.shape[0]
    pos_f = positions.reshape(N, 1).astype(jnp.float32)

    div = np.exp(np.arange(0, d_half, 2, dtype=np.float64)
                 * -(np.log(10000.0) / d_half))
    df = jnp.asarray(np.repeat(div / (2.0 * np.pi), 2).reshape(1, d_half),
                     dtype=jnp.float32)
    ph = jnp.asarray(np.tile(np.array([0.5, 0.75]), d_half // 2)
                     .reshape(1, d_half), dtype=jnp.float32)
    wt = W.T
    b2 = b.reshape(1, d_half)
    g2 = gamma.reshape(1, d_half)
    be2 = beta.reshape(1, d_half)

    T = _TOK_BLOCK
    grid = (N // T,)
    out = pl.pallas_call(
        _pe_kernel,
        grid=grid,
        in_specs=[
            pl.BlockSpec((T, 1), lambda i: (i, 0)),
            pl.BlockSpec((1, d_half), lambda i: (0, 0)),
            pl.BlockSpec((1, d_half), lambda i: (0, 0)),
            pl.BlockSpec((3, d_half), lambda i: (0, 0)),
            pl.BlockSpec((1, d_half), lambda i: (0, 0)),
            pl.BlockSpec((1, d_half), lambda i: (0, 0)),
            pl.BlockSpec((1, d_half), lambda i: (0, 0)),
        ],
        out_specs=pl.BlockSpec((T, 2 * d_half), lambda i: (i, 0)),
        out_shape=jax.ShapeDtypeStruct((N, 2 * d_half), jnp.float32),
    )(pos_f, df, ph, wt, b2, g2, be2)
    return out.reshape(B, S, 2 * d_half)
